# baseline (speedup 1.0000x reference)
import jax
import jax.numpy as jnp
from jax import lax
from jax.experimental import pallas as pl
from jax.experimental.pallas import tpu as pltpu
from jax.experimental.pallas import tpu_sc as plsc

NC = 2
NS = 16
NW = NC * NS
LANES = 16

BATCH = 16384
LABELS = 50
EMBED = 64
EPAD = 128

BW = BATCH // NW
CHUNK = 128
NSUB = BW // CHUNK
NCHUNK = NSUB * LABELS
NBUF = 4
PB = 1024


def _slice_body(x_ref, o_ref):
  o_ref[...] = x_ref[:, pl.ds(0, EMBED)]


def _sc_body(idx_hbm, w_hbm, out_hbm, idx_v, acc_v,
             b0, b1, b2, b3, s0, s1, s2, s3):
  bufs = (b0, b1, b2, b3)
  sems = (s0, s1, s2, s3)

  wid = lax.axis_index("s") * NC + lax.axis_index("c")

  pltpu.sync_copy(idx_hbm.at[:, pl.ds(wid * BW, BW)], idx_v)

  def idx_slice(t):
    return idx_v.at[t >> 2, pl.ds((t & (NSUB - 1)) * CHUNK, CHUNK)]

  for b in range(NBUF):
    pltpu.async_copy(w_hbm.at[idx_slice(b)], bufs[b], sems[b])

  zero = jnp.zeros((LANES,), jnp.float32)

  @pl.loop(0, BW, unroll=4)
  def _zero(r):
    for c in range(EMBED // LANES):
      acc_v[r, pl.ds(c * LANES, LANES)] = zero

  @pl.loop(0, NCHUNK, step=NBUF)
  def _main(t):
    for b in range(NBUF):
      tt = t + b
      buf = bufs[b]
      sem = sems[b]
      pltpu.make_async_copy(w_hbm.at[idx_slice(tt)], buf, sem).wait()

      base = (tt & (NSUB - 1)) * CHUNK

      @pl.loop(0, CHUNK, unroll=8)
      def _accum(i):
        row = base + i
        for c in range(EMBED // LANES):
          v = buf[i, pl.ds(c * LANES, LANES)]
          plsc.addupdate(acc_v.at[row, pl.ds(c * LANES, LANES)], v)

      nxt = tt + NBUF

      @pl.when(nxt < NCHUNK)
      def _():
        pltpu.async_copy(w_hbm.at[idx_slice(nxt)], buf, sem)

  pltpu.sync_copy(acc_v, out_hbm.at[pl.ds(wid * BW, BW), pl.ds(0, EMBED)])


@jax.jit
def _run(idx_t, weight):
  mesh = plsc.VectorSubcoreMesh(
      core_axis_name="c", subcore_axis_name="s",
      num_cores=NC, num_subcores=NS)
  f = pl.kernel(
      _sc_body,
      out_type=jax.ShapeDtypeStruct((BATCH, EPAD), jnp.float32),
      mesh=mesh,
      scratch_types=[
          pltpu.VMEM((LABELS, BW), jnp.int32),
          pltpu.VMEM((BW, EMBED), jnp.float32),
      ] + [pltpu.VMEM((CHUNK, EPAD), jnp.float32)] * NBUF
        + [pltpu.SemaphoreType.DMA] * NBUF,
      compiler_params=pltpu.CompilerParams(use_tc_tiling_on_sc=False,
                                           needs_layout_passes=False),
  )
  wide = f(idx_t, weight)
  return pl.pallas_call(
      _slice_body,
      grid=(BATCH // PB,),
      in_specs=[pl.BlockSpec((PB, EPAD), lambda i: (i, 0))],
      out_specs=pl.BlockSpec((PB, EMBED), lambda i: (i, 0)),
      out_shape=jax.ShapeDtypeStruct((BATCH, EMBED), jnp.float32),
  )(wide)


def kernel(inputs, weight):
  wpad = jnp.pad(weight, ((0, 0), (0, EPAD - EMBED)))
  return _run(inputs.astype(jnp.int32).T, wpad)


# speedup vs baseline: 1.2364x; 1.2364x over previous
import jax
import jax.numpy as jnp
from jax import lax
from jax.experimental import pallas as pl
from jax.experimental.pallas import tpu as pltpu
from jax.experimental.pallas import tpu_sc as plsc

NC = 2
NS = 16
NW = NC * NS
LANES = 16

BATCH = 16384
LABELS = 50
EMBED = 64
EPAD = 128

BW = BATCH // NW
CHUNK = 128
NSUB = BW // CHUNK
NCHUNK = NSUB * LABELS
NBUF = 4
PB = 1024


def _slice_body(x_ref, o_ref):
  o_ref[...] = x_ref[:, pl.ds(0, EMBED)]


def _sc_body(idx_hbm, w_hbm, out_hbm, idx_v, acc_v,
             b0, b1, b2, b3, s0, s1, s2, s3):
  bufs = (b0, b1, b2, b3)
  sems = (s0, s1, s2, s3)

  wid = lax.axis_index("s") * NC + lax.axis_index("c")

  pltpu.sync_copy(idx_hbm.at[:, pl.ds(wid * BW, BW)], idx_v)

  def idx_slice(t):
    return idx_v.at[t >> 2, pl.ds((t & (NSUB - 1)) * CHUNK, CHUNK)]

  for b in range(NBUF):
    pltpu.async_copy(w_hbm.at[idx_slice(b)], bufs[b], sems[b])

  zero = jnp.zeros((LANES,), jnp.float32)

  @pl.loop(0, BW, unroll=4)
  def _zero(r):
    for c in range(EMBED // LANES):
      acc_v[r, pl.ds(c * LANES, LANES)] = zero

  @pl.loop(0, NCHUNK, step=NBUF)
  def _main(t):
    for b in range(NBUF):
      tt = t + b
      buf = bufs[b]
      sem = sems[b]
      pltpu.make_async_copy(w_hbm.at[idx_slice(tt)], buf, sem).wait()

      base = (tt & (NSUB - 1)) * CHUNK

      @pl.loop(0, CHUNK, unroll=8)
      def _accum(i):
        row = base + i
        for c in range(EMBED // LANES):
          v = buf[i, pl.ds(c * LANES, LANES)]
          plsc.addupdate(acc_v.at[row, pl.ds(c * LANES, LANES)], v)

      nxt = tt + NBUF

      @pl.when(nxt < NCHUNK)
      def _():
        pltpu.async_copy(w_hbm.at[idx_slice(nxt)], buf, sem)

  pltpu.sync_copy(acc_v, out_hbm.at[pl.ds(wid * BW, BW)])


@jax.jit
def _run(idx_t, weight):
  mesh = plsc.VectorSubcoreMesh(
      core_axis_name="c", subcore_axis_name="s",
      num_cores=NC, num_subcores=NS)
  f = pl.kernel(
      _sc_body,
      out_type=jax.ShapeDtypeStruct((BATCH, EMBED), jnp.float32),
      mesh=mesh,
      scratch_types=[
          pltpu.VMEM((LABELS, BW), jnp.int32),
          pltpu.VMEM((BW, EMBED), jnp.float32),
      ] + [pltpu.VMEM((CHUNK, EMBED), jnp.float32)] * NBUF
        + [pltpu.SemaphoreType.DMA] * NBUF,
      compiler_params=pltpu.CompilerParams(use_tc_tiling_on_sc=False,
                                           needs_layout_passes=False),
  )
  return f(idx_t, weight)


def kernel(inputs, weight):
  return _run(inputs.astype(jnp.int32).T, weight)
